# baseline (speedup 1.0000x reference)
import jax
import jax.numpy as jnp
from jax.experimental import pallas as pl
from jax.experimental.pallas import tpu as pltpu

_N_ID = 100000
_N_CLS = 1000
_ROWS = 256
_CB = 7168
_NC = (_N_ID + _CB - 1) // _CB
_LANES = 128
_CHUNKS = _CB // _LANES

_TINY = 1e-30


def _xlogy_t(t):
    safe = jnp.where(t > 0, t, 1.0)
    return jnp.where(t > 0, t * jnp.log(safe), 0.0)


def _icp_kernel(si_ref, ti_ref, sc_ref, tc_ref, tp_ref, t_ref,
                sp_e0_ref, sp_e1_ref, sp_o0_ref, sp_o1_ref,
                tp_e0_ref, tp_e1_ref, tp_o0_ref, tp_o1_ref,
                tpol_ref, out_ref,
                m_l, z_l, s1_l, s2_l, s3_l, si01_l):
    j = pl.program_id(0)

    @pl.when(j == 0)
    def _init():
        m_l[:] = jnp.full((_ROWS, _LANES), -jnp.inf, dtype=jnp.float32)
        z_l[:] = jnp.zeros((_ROWS, _LANES), dtype=jnp.float32)
        s1_l[:] = jnp.zeros((_ROWS, _LANES), dtype=jnp.float32)
        s2_l[:] = jnp.zeros((_ROWS, _LANES), dtype=jnp.float32)
        s3_l[:] = jnp.zeros((_ROWS, _LANES), dtype=jnp.float32)
        si01_l[:] = si_ref[:, 0:2]

    def accum(masked):
        x = si_ref[:]
        tt = ti_ref[:]
        if masked:
            base = j * _CB
            iota = jax.lax.broadcasted_iota(jnp.int32, (_ROWS, _LANES), 1)
            valids = [(iota + (base + k * _LANES)) < _N_ID
                      for k in range(_CHUNKS)]

        bm = jnp.full((_ROWS, _LANES), -jnp.inf, dtype=jnp.float32)
        for k in range(_CHUNKS):
            c = x[:, k * _LANES:(k + 1) * _LANES]
            if masked:
                c = jnp.where(valids[k], c, -jnp.inf)
            bm = jnp.maximum(bm, c)

        m_old = m_l[:]
        m_new = jnp.maximum(m_old, bm)
        z = z_l[:] * jnp.exp(m_old - m_new)
        s1 = s1_l[:]
        s2 = s2_l[:]
        s3 = s3_l[:]

        for k in range(_CHUNKS):
            sl = slice(k * _LANES, (k + 1) * _LANES)
            c = x[:, sl]
            tv = tt[:, sl]
            if masked:
                c = jnp.where(valids[k], c, -jnp.inf)
                tv = jnp.where(valids[k], tv, 0.0)
                z = z + jnp.exp(c - m_new)
                s2 = s2 + tv * jnp.where(valids[k], x[:, sl], 0.0)
            else:
                z = z + jnp.exp(c - m_new)
                s2 = s2 + tv * c
            s1 = s1 + tv
            s3 = s3 + tv * jnp.log(jnp.maximum(tv, _TINY))

        m_l[:] = m_new
        z_l[:] = z
        s1_l[:] = s1
        s2_l[:] = s2
        s3_l[:] = s3
        return m_new, z, s1, s2, s3

    @pl.when(j < _NC - 1)
    def _fast():
        accum(masked=False)

    @pl.when(j == _NC - 1)
    def _last():
        f32 = jnp.float32
        m_new, z, s1, s2, s3 = accum(masked=True)
        m_f = jnp.max(m_new, axis=1, keepdims=True)
        zz = jnp.sum(z * jnp.exp(m_new - m_f), axis=1, keepdims=True)
        lz_id = m_f + jnp.log(zz)
        s1s = jnp.sum(s1, axis=1, keepdims=True)
        s2s = jnp.sum(s2, axis=1, keepdims=True)
        s3s = jnp.sum(s3, axis=1, keepdims=True)
        klid_sum = jnp.sum(s3s - s2s + lz_id * s1s)

        t0 = t_ref[:, 0:1]
        t1 = t_ref[:, 1:2]
        s01 = si01_l[:]
        pick_i = jnp.where(t0 == 0, s01[:, 0:1], s01[:, 1:2])
        cei_sum = jnp.sum(lz_id - pick_i)

        x = sc_ref[:]
        tcv = tc_ref[:]
        m = jnp.max(x, axis=1, keepdims=True)
        zc = jnp.sum(jnp.exp(x - m), axis=1, keepdims=True)
        lzc = m + jnp.log(zc)
        c1 = jnp.sum(tcv, axis=1, keepdims=True)
        c2 = jnp.sum(tcv * x, axis=1, keepdims=True)
        c3 = jnp.sum(_xlogy_t(tcv), axis=1, keepdims=True)
        klc_sum = jnp.sum(c3 - c2 + lzc * c1)

        iota_c = jax.lax.broadcasted_iota(jnp.int32, (_ROWS, _N_CLS), 1)
        mt = jnp.max(tcv, axis=1, keepdims=True)
        amax_c = jnp.min(jnp.where(tcv == mt, iota_c, _N_CLS), axis=1,
                         keepdims=True)
        cnt_cls = jnp.sum((amax_c == t1).astype(f32))
        pick_c = jnp.where(t1 == 0, x[:, 0:1], x[:, 1:2])
        cec_sum = jnp.sum(lzc - pick_c)

        tpv = tp_ref[:]
        mtp = jnp.max(tpv, axis=1, keepdims=True)
        iota_p = jax.lax.broadcasted_iota(jnp.int32, (_ROWS, 28), 1)
        amax_tp = jnp.min(jnp.where(tpv == mtp, iota_p, 28), axis=1,
                          keepdims=True)
        cnt_id = jnp.sum((amax_tp == t0).astype(f32))

        sd0 = sp_o0_ref[:] - sp_e0_ref[:]
        sd1 = sp_o1_ref[:] - sp_e1_ref[:]
        td0 = tp_o0_ref[:] - tp_e0_ref[:]
        td1 = tp_o1_ref[:] - tp_e1_ref[:]
        mx = jnp.maximum(sd0, sd1)
        lse = mx + jnp.log(jnp.exp(sd0 - mx) + jnp.exp(sd1 - mx))
        lp0 = sd0 - lse
        lp1 = sd1 - lse
        klp = (_xlogy_t(td0) - td0 * lp0) + (_xlogy_t(td1) - td1 * lp1)
        klp_col = jnp.sum(klp, axis=0, keepdims=True)
        g = tpol_ref[:]
        cep_col = jnp.sum(jnp.where(g == 0, -lp0, -lp1), axis=0,
                          keepdims=True)
        amax2 = jnp.where(tp_o1_ref[:] > tp_o0_ref[:], 1, 0)
        pw_col = jnp.sum((g == amax2).astype(f32), axis=0,
                         keepdims=True) / 128.0

        id_w = (_N_ID * cnt_id / 256.0 - 1.0) / (_N_ID - 1.0)
        cls_w = (_N_CLS * cnt_cls / 256.0 - 1.0) / (_N_CLS - 1.0)
        kl_id = id_w * klid_sum / (256.0 * _N_ID)
        kl_cls = cls_w * 0.5 * klc_sum / (256.0 * _N_CLS)
        kl_pol = jnp.sum(pw_col * klp_col) * (0.001 / 256.0)
        ce_id = cei_sum / (256.0 * _N_ID)
        ce_cls = 0.5 * cec_sum / (256.0 * _N_CLS)
        ce_pol = jnp.sum(pw_col * cep_col) * (0.001 / 256.0)
        total = kl_id + kl_cls + kl_pol + ce_id + ce_cls + ce_pol
        out_ref[:] = jnp.broadcast_to(total, (1, 1))


@jax.jit
def kernel(student_identity, student_classes, student_policy,
           teacher_identity, teacher_classes, teacher_policy, targets):
    si = student_identity
    sc = student_classes
    sp = student_policy
    ti = teacher_identity
    tc = teacher_classes
    tp = teacher_policy
    t = targets.reshape(-1, targets.shape[-1]).astype(jnp.int32)

    sp_e = sp[0::2]
    sp_o = sp[1::2]
    tp_e = tp[0::2]
    tp_o = tp[1::2]
    small = (sc, tc, tp, t,
             sp_e[:, 0::2], sp_e[:, 1::2], sp_o[:, 0::2], sp_o[:, 1::2],
             tp_e[:, 0::2], tp_e[:, 1::2], tp_o[:, 0::2], tp_o[:, 1::2],
             t[1::2, 2:])

    const_specs = [
        pl.BlockSpec(a.shape, lambda i: (0, 0)) for a in small
    ]
    out = pl.pallas_call(
        _icp_kernel,
        grid=(_NC,),
        in_specs=[
            pl.BlockSpec((_ROWS, _CB), lambda i: (0, i)),
            pl.BlockSpec((_ROWS, _CB), lambda i: (0, i)),
        ] + const_specs,
        out_specs=pl.BlockSpec((1, 1), lambda i: (0, 0)),
        out_shape=jax.ShapeDtypeStruct((1, 1), jnp.float32),
        scratch_shapes=[pltpu.VMEM((_ROWS, _LANES), jnp.float32)] * 5
        + [pltpu.VMEM((_ROWS, 2), jnp.float32)],
    )(si, ti, *small)
    return out[0, 0]


# speedup vs baseline: 1.0229x; 1.0229x over previous
import jax
import jax.numpy as jnp
from jax.experimental import pallas as pl
from jax.experimental.pallas import tpu as pltpu

_N_ID = 100000
_N_CLS = 1000
_ROWS = 256
_CB = 7168
_NC = (_N_ID + _CB - 1) // _CB
_LANES = 128
_CHUNKS = _CB // _LANES
_RBS = 256


def _xlogy_t(t):
    safe = jnp.where(t > 0, t, 1.0)
    return jnp.where(t > 0, t * jnp.log(safe), 0.0)


_TINY = 1e-30


def _id_stats_kernel(si_ref, ti_ref, akl_ref, lz_ref,
                     m_l, z_l, s1_l, s2_l, s3_l):
    j = pl.program_id(1)

    @pl.when(j == 0)
    def _init():
        m_l[:] = jnp.full((_RBS, _LANES), -jnp.inf, dtype=jnp.float32)
        z_l[:] = jnp.zeros((_RBS, _LANES), dtype=jnp.float32)
        s1_l[:] = jnp.zeros((_RBS, _LANES), dtype=jnp.float32)
        s2_l[:] = jnp.zeros((_RBS, _LANES), dtype=jnp.float32)
        s3_l[:] = jnp.zeros((_RBS, _LANES), dtype=jnp.float32)

    def accum(masked):
        x = si_ref[:]
        tt = ti_ref[:]
        if masked:
            base = j * _CB
            iota = jax.lax.broadcasted_iota(jnp.int32, (_RBS, _LANES), 1)
            valids = [(iota + (base + k * _LANES)) < _N_ID
                      for k in range(_CHUNKS)]

        bm = jnp.full((_RBS, _LANES), -jnp.inf, dtype=jnp.float32)
        for k in range(_CHUNKS):
            c = x[:, k * _LANES:(k + 1) * _LANES]
            if masked:
                c = jnp.where(valids[k], c, -jnp.inf)
            bm = jnp.maximum(bm, c)

        m_old = m_l[:]
        m_new = jnp.maximum(m_old, bm)
        z = z_l[:] * jnp.exp(m_old - m_new)
        s1 = s1_l[:]
        s2 = s2_l[:]
        s3 = s3_l[:]

        for k in range(_CHUNKS):
            sl = slice(k * _LANES, (k + 1) * _LANES)
            c = x[:, sl]
            tv = tt[:, sl]
            if masked:
                c = jnp.where(valids[k], c, -jnp.inf)
                tv = jnp.where(valids[k], tv, 0.0)
                e = jnp.exp(c - m_new)
                z = z + e
                s2 = s2 + tv * jnp.where(valids[k], x[:, sl], 0.0)
            else:
                z = z + jnp.exp(c - m_new)
                s2 = s2 + tv * c
            s1 = s1 + tv
            s3 = s3 + tv * jnp.log(jnp.maximum(tv, _TINY))

        m_l[:] = m_new
        z_l[:] = z
        s1_l[:] = s1
        s2_l[:] = s2
        s3_l[:] = s3
        return m_new, z, s1, s2, s3

    @pl.when(j < _NC - 1)
    def _fast():
        accum(masked=False)

    @pl.when(j == _NC - 1)
    def _last():
        m_new, z, s1, s2, s3 = accum(masked=True)
        m_f = jnp.max(m_new, axis=1, keepdims=True)
        zz = jnp.sum(z * jnp.exp(m_new - m_f), axis=1, keepdims=True)
        lz = m_f + jnp.log(zz)
        s1s = jnp.sum(s1, axis=1, keepdims=True)
        s2s = jnp.sum(s2, axis=1, keepdims=True)
        s3s = jnp.sum(s3, axis=1, keepdims=True)
        akl_ref[:] = s3s - s2s + lz * s1s
        lz_ref[:] = lz


def _combine_kernel(sc_ref, tc_ref, tp_ref, t_ref, si01_ref,
                    sp_e0_ref, sp_e1_ref, sp_o0_ref, sp_o1_ref,
                    tp_e0_ref, tp_e1_ref, tp_o0_ref, tp_o1_ref,
                    tpol_ref, akl_ref, lz_ref, out_ref):
    f32 = jnp.float32
    x = sc_ref[:]
    tcv = tc_ref[:]
    m = jnp.max(x, axis=1, keepdims=True)
    z = jnp.sum(jnp.exp(x - m), axis=1, keepdims=True)
    lzc = m + jnp.log(z)
    s1 = jnp.sum(tcv, axis=1, keepdims=True)
    s2 = jnp.sum(tcv * x, axis=1, keepdims=True)
    s3 = jnp.sum(_xlogy_t(tcv), axis=1, keepdims=True)
    klc_sum = jnp.sum(s3 - s2 + lzc * s1)

    t0 = t_ref[:, 0:1]
    t1 = t_ref[:, 1:2]
    iota_c = jax.lax.broadcasted_iota(jnp.int32, (_ROWS, _N_CLS), 1)
    mt = jnp.max(tcv, axis=1, keepdims=True)
    amax_c = jnp.min(jnp.where(tcv == mt, iota_c, _N_CLS), axis=1, keepdims=True)
    cnt_cls = jnp.sum((amax_c == t1).astype(f32))
    pick_c = jnp.where(t1 == 0, x[:, 0:1], x[:, 1:2])
    cec_sum = jnp.sum(lzc - pick_c)

    klid_sum = jnp.sum(akl_ref[:])
    lz_id = lz_ref[:]
    s01 = si01_ref[:]
    pick_i = jnp.where(t0 == 0, s01[:, 0:1], s01[:, 1:2])
    cei_sum = jnp.sum(lz_id - pick_i)

    tpv = tp_ref[:]
    mtp = jnp.max(tpv, axis=1, keepdims=True)
    iota_p = jax.lax.broadcasted_iota(jnp.int32, (_ROWS, 28), 1)
    amax_tp = jnp.min(jnp.where(tpv == mtp, iota_p, 28), axis=1, keepdims=True)
    cnt_id = jnp.sum((amax_tp == t0).astype(f32))

    sd0 = sp_o0_ref[:] - sp_e0_ref[:]
    sd1 = sp_o1_ref[:] - sp_e1_ref[:]
    td0 = tp_o0_ref[:] - tp_e0_ref[:]
    td1 = tp_o1_ref[:] - tp_e1_ref[:]
    mx = jnp.maximum(sd0, sd1)
    lse = mx + jnp.log(jnp.exp(sd0 - mx) + jnp.exp(sd1 - mx))
    lp0 = sd0 - lse
    lp1 = sd1 - lse
    klp = (_xlogy_t(td0) - td0 * lp0) + (_xlogy_t(td1) - td1 * lp1)
    klp_col = jnp.sum(klp, axis=0, keepdims=True)
    g = tpol_ref[:]
    cep_col = jnp.sum(jnp.where(g == 0, -lp0, -lp1), axis=0, keepdims=True)
    amax2 = jnp.where(tp_o1_ref[:] > tp_o0_ref[:], 1, 0)
    pw_col = jnp.sum((g == amax2).astype(f32), axis=0, keepdims=True) / 128.0

    id_w = (_N_ID * cnt_id / 256.0 - 1.0) / (_N_ID - 1.0)
    cls_w = (_N_CLS * cnt_cls / 256.0 - 1.0) / (_N_CLS - 1.0)
    kl_id = id_w * klid_sum / (256.0 * _N_ID)
    kl_cls = cls_w * 0.5 * klc_sum / (256.0 * _N_CLS)
    kl_pol = jnp.sum(pw_col * klp_col) * (0.001 / 256.0)
    ce_id = cei_sum / (256.0 * _N_ID)
    ce_cls = 0.5 * cec_sum / (256.0 * _N_CLS)
    ce_pol = jnp.sum(pw_col * cep_col) * (0.001 / 256.0)
    total = kl_id + kl_cls + kl_pol + ce_id + ce_cls + ce_pol
    out_ref[:] = jnp.broadcast_to(total, (1, 1))


@jax.jit
def kernel(student_identity, student_classes, student_policy,
           teacher_identity, teacher_classes, teacher_policy, targets):
    si = student_identity
    sc = student_classes
    sp = student_policy
    ti = teacher_identity
    tc = teacher_classes
    tp = teacher_policy
    t = targets.reshape(-1, targets.shape[-1]).astype(jnp.int32)

    akl, lz = pl.pallas_call(
        _id_stats_kernel,
        grid=(_ROWS // _RBS, _NC),
        in_specs=[
            pl.BlockSpec((_RBS, _CB), lambda i, j: (i, j)),
            pl.BlockSpec((_RBS, _CB), lambda i, j: (i, j)),
        ],
        out_specs=[
            pl.BlockSpec((_RBS, 1), lambda i, j: (i, 0)),
            pl.BlockSpec((_RBS, 1), lambda i, j: (i, 0)),
        ],
        out_shape=[
            jax.ShapeDtypeStruct((_ROWS, 1), jnp.float32),
            jax.ShapeDtypeStruct((_ROWS, 1), jnp.float32),
        ],
        scratch_shapes=[pltpu.VMEM((_RBS, _LANES), jnp.float32)] * 5,
    )(si, ti)

    si01 = jax.lax.slice(si, (0, 0), (_ROWS, 2))
    sp_e = sp[0::2]
    sp_o = sp[1::2]
    tp_e = tp[0::2]
    tp_o = tp[1::2]
    args = (sc, tc, tp, t, si01,
            sp_e[:, 0::2], sp_e[:, 1::2], sp_o[:, 0::2], sp_o[:, 1::2],
            tp_e[:, 0::2], tp_e[:, 1::2], tp_o[:, 0::2], tp_o[:, 1::2],
            t[1::2, 2:], akl, lz)

    out = pl.pallas_call(
        _combine_kernel,
        out_shape=jax.ShapeDtypeStruct((1, 1), jnp.float32),
    )(*args)
    return out[0, 0]
